# baseline (speedup 1.0000x reference)
import functools

import jax
import jax.numpy as jnp
from jax import lax
from jax.experimental import pallas as pl
from jax.experimental.pallas import tpu as pltpu
from jax.experimental.pallas import tpu_sc as plsc

B = 16
S = 2048
D = 1024
NC = 2
NS = 16
NW = NC * NS
HALF = S // 2
R = 64
NCHUNK = HALF // R


def _pos_embed_body(x_hbm, pose_hbm, out_hbm, xrow_v, buf_v):
    c = lax.axis_index("c")
    s = lax.axis_index("s")
    wid = c * NS + s
    i = wid // 2
    h = wid % 2
    j0 = h * HALF

    pltpu.sync_copy(x_hbm.at[pl.ds(i * S, S)], xrow_v)

    def scan_body(k, acc):
        v = xrow_v[pl.ds(k * 16, 16)]
        idx = lax.iota(jnp.int32, 16) + k * 16
        cand = jnp.where(v != 0, idx, S)
        return jnp.minimum(acc, cand)

    acc = lax.fori_loop(0, S // 16, scan_body, jnp.full((16,), S, jnp.int32))
    m = jnp.int32(S)
    for l in range(16):
        m = jnp.minimum(m, acc[l])
    fi = jnp.where(m >= S, 0, m)

    start = j0 - fi + S

    src_off = pl.multiple_of(start * D, 8)
    dst_off = pl.multiple_of((i * S + j0) * D, 8)
    pltpu.sync_copy(
        pose_hbm.at[pl.ds(src_off, HALF * D)],
        out_hbm.at[pl.ds(dst_off, HALF * D)],
    )


_pos_embed = functools.partial(
    pl.kernel,
    out_type=jax.ShapeDtypeStruct((B * S * D,), jnp.float32),
    mesh=plsc.VectorSubcoreMesh(core_axis_name="c", subcore_axis_name="s"),
    scratch_types=[
        pltpu.VMEM((S,), jnp.int32),
        pltpu.VMEM((R * D,), jnp.float32),
    ],
)(_pos_embed_body)


@jax.jit
def kernel(x, pos):
    pos_ext = jnp.concatenate([jnp.zeros((S, D), pos.dtype), pos], axis=0)
    out = _pos_embed(x.astype(jnp.int32).reshape(-1), pos_ext.reshape(-1))
    return out.reshape(B, S, D)


# speedup vs baseline: 14.7192x; 14.7192x over previous
import functools

import jax
import jax.numpy as jnp
from jax import lax
from jax.experimental import pallas as pl
from jax.experimental.pallas import tpu as pltpu
from jax.experimental.pallas import tpu_sc as plsc

B = 16
S = 2048
D = 1024
NC = 2
NS = 16
NW = NC * NS
HALF = S // 2
R = 16
NBUF = 4
NCHUNK = HALF // R


def _pos_embed_body(x_hbm, pose_hbm, out_hbm, xrow_v,
                    b0, b1, b2, b3, si0, si1, si2, si3, so0, so1, so2, so3):
    bufs = (b0, b1, b2, b3)
    sins = (si0, si1, si2, si3)
    souts = (so0, so1, so2, so3)
    c = lax.axis_index("c")
    s = lax.axis_index("s")
    wid = c * NS + s
    i = wid // 2
    h = wid % 2
    j0 = h * HALF

    pltpu.sync_copy(x_hbm.at[pl.ds(i * S, S)], xrow_v)

    def scan_body(k, acc):
        v = xrow_v[pl.ds(k * 16, 16)]
        idx = lax.iota(jnp.int32, 16) + k * 16
        cand = jnp.where(v != 0, idx, S)
        return jnp.minimum(acc, cand)

    acc = lax.fori_loop(0, S // 16, scan_body, jnp.full((16,), S, jnp.int32))
    m = jnp.int32(S)
    for l in range(16):
        m = jnp.minimum(m, acc[l])
    fi = jnp.where(m >= S, 0, m)

    start = j0 - fi + S

    def in_copy(t, b):
        off = pl.multiple_of((start + t * R) * D, 8)
        return pltpu.make_async_copy(
            pose_hbm.at[pl.ds(off, R * D)], bufs[b], sins[b])

    def out_copy(t, b):
        off = pl.multiple_of((i * S + j0 + t * R) * D, 8)
        return pltpu.make_async_copy(
            bufs[b], out_hbm.at[pl.ds(off, R * D)], souts[b])

    for b in range(NBUF):
        in_copy(b, b).start()

    def group(g, _):
        for b in range(NBUF):
            t = g * NBUF + b
            in_copy(t, b).wait()
            out_copy(t, b).start()
            nxt = t + NBUF

            @pl.when(nxt < NCHUNK)
            def _():
                out_copy(t, b).wait()
                in_copy(nxt, b).start()
        return 0

    lax.fori_loop(0, NCHUNK // NBUF, group, 0)

    for b in range(NBUF):
        out_copy(NCHUNK - NBUF + b, b).wait()


_pos_embed = functools.partial(
    pl.kernel,
    out_type=jax.ShapeDtypeStruct((B * S * D,), jnp.float32),
    mesh=plsc.VectorSubcoreMesh(core_axis_name="c", subcore_axis_name="s"),
    scratch_types=(
        [pltpu.VMEM((S,), jnp.int32)]
        + [pltpu.VMEM((R * D,), jnp.float32) for _ in range(NBUF)]
        + [pltpu.SemaphoreType.DMA for _ in range(2 * NBUF)]
    ),
)(_pos_embed_body)


@jax.jit
def kernel(x, pos):
    pos_ext = jnp.concatenate([jnp.zeros((S, D), pos.dtype), pos], axis=0)
    out = _pos_embed(x.astype(jnp.int32).reshape(-1), pos_ext.reshape(-1))
    return out.reshape(B, S, D)


# speedup vs baseline: 24.1544x; 1.6410x over previous
import functools

import jax
import jax.numpy as jnp
from jax import lax
from jax.experimental import pallas as pl
from jax.experimental.pallas import tpu as pltpu
from jax.experimental.pallas import tpu_sc as plsc

B = 16
S = 2048
D = 1024
NC = 2
NS = 16
NW = NC * NS
HALF = S // 2
R = 16
NBUF = 4
NCHUNK = HALF // R


def _pos_embed_body(x_hbm, pose_hbm, out_hbm, xrow_v,
                    b0, b1, b2, b3, si0, si1, si2, si3, so0, so1, so2, so3):
    bufs = (b0, b1, b2, b3)
    sins = (si0, si1, si2, si3)
    souts = (so0, so1, so2, so3)
    c = lax.axis_index("c")
    s = lax.axis_index("s")
    wid = c * NS + s
    i = wid // 2
    h = wid % 2
    j0 = h * HALF

    pltpu.sync_copy(x_hbm.at[pl.ds(i * S, S)], xrow_v)

    def scan_body(k, acc):
        v = xrow_v[pl.ds(k * 16, 16)]
        idx = lax.iota(jnp.int32, 16) + k * 16
        cand = jnp.where(v != 0, idx, S)
        return jnp.minimum(acc, cand)

    acc = lax.fori_loop(0, S // 16, scan_body, jnp.full((16,), S, jnp.int32))
    m = jnp.int32(S)
    for l in range(16):
        m = jnp.minimum(m, acc[l])
    fi = jnp.where(m >= S, 0, m)

    start = j0 - fi + S

    def in_copies(t, b):
        off = pl.multiple_of((start + t * R) * D, 8)
        return [
            pltpu.make_async_copy(
                pose_hbm.at[pl.ds(off + r * D, D)], bufs[b].at[r], sins[b])
            for r in range(R)
        ]

    def out_copy(t, b):
        row = pl.multiple_of(i * S + j0 + t * R, 8)
        return pltpu.make_async_copy(
            bufs[b], out_hbm.at[pl.ds(row, R), :], souts[b])

    for b in range(NBUF):
        for cp in in_copies(b, b):
            cp.start()

    def group(g, _):
        for b in range(NBUF):
            t = g * NBUF + b
            for cp in in_copies(t, b):
                cp.wait()
            out_copy(t, b).start()
            nxt = t + NBUF

            @pl.when(nxt < NCHUNK)
            def _():
                out_copy(t, b).wait()
                for cp in in_copies(nxt, b):
                    cp.start()
        return 0

    lax.fori_loop(0, NCHUNK // NBUF, group, 0)

    for b in range(NBUF):
        out_copy(NCHUNK - NBUF + b, b).wait()


_pos_embed = functools.partial(
    pl.kernel,
    out_type=jax.ShapeDtypeStruct((B * S, D), jnp.float32),
    mesh=plsc.VectorSubcoreMesh(core_axis_name="c", subcore_axis_name="s"),
    scratch_types=(
        [pltpu.VMEM((S,), jnp.int32)]
        + [pltpu.VMEM((R, D), jnp.float32) for _ in range(NBUF)]
        + [pltpu.SemaphoreType.DMA for _ in range(2 * NBUF)]
    ),
)(_pos_embed_body)


@jax.jit
def kernel(x, pos):
    pos_ext = jnp.concatenate([jnp.zeros((S, D), pos.dtype), pos], axis=0)
    out = _pos_embed(x.astype(jnp.int32).reshape(-1), pos_ext.reshape(-1))
    return out.reshape(B, S, D)


# speedup vs baseline: 30.9686x; 1.2821x over previous
import functools

import jax
import jax.numpy as jnp
from jax import lax
from jax.experimental import pallas as pl
from jax.experimental.pallas import tpu as pltpu
from jax.experimental.pallas import tpu_sc as plsc

B = 16
S = 2048
D = 1024
NC = 2
NS = 16
HALF = S // 2
R = 16
NBUF = 4
NCHUNK = HALF // R


def _pos_embed_body(x_hbm, pos_hbm, out_hbm, xall_v,
                    i0, i1, i2, i3, b0, b1, b2, b3,
                    si0, si1, si2, si3, so0, so1, so2, so3):
    idxs = (i0, i1, i2, i3)
    bufs = (b0, b1, b2, b3)
    sins = (si0, si1, si2, si3)
    souts = (so0, so1, so2, so3)

    c = lax.axis_index("c")
    s = lax.axis_index("s")
    wid = c * NS + s
    i = wid // 2
    h = wid % 2
    j0 = h * HALF

    pltpu.sync_copy(x_hbm, xall_v)

    def scan_body(k, acc):
        v = xall_v[i, pl.ds(k * 16, 16)]
        cand = jnp.where(v != 0, lax.iota(jnp.int32, 16) + k * 16, S)
        return jnp.minimum(acc, cand)

    acc = lax.fori_loop(0, S // 16, scan_body, jnp.full((16,), S, jnp.int32))
    m = jnp.int32(S)
    for l in range(16):
        m = jnp.minimum(m, acc[l])
    fi = jnp.where(m >= S, 0, m)

    rel0 = j0 - fi

    def in_copy(t, b):
        base = rel0 + t * R
        idxs[b][...] = jnp.clip(lax.iota(jnp.int32, 16) + base, 0, S - 1)
        return pltpu.make_async_copy(pos_hbm.at[idxs[b]], bufs[b], sins[b])

    def in_wait(b):
        pltpu.make_async_copy(pos_hbm.at[idxs[b]], bufs[b], sins[b]).wait()

    def zero_fix(t, b):
        nz = jnp.clip(fi - (j0 + t * R), 0, R)

        @pl.when(nz > 0)
        def _():
            def zrow(r, _):
                for col in range(D // 16):
                    bufs[b][r, pl.ds(col * 16, 16)] = jnp.zeros(
                        (16,), jnp.float32)
                return 0
            lax.fori_loop(0, nz, zrow, 0)

    def out_copy(t, b):
        row = pl.multiple_of(i * S + j0 + t * R, 8)
        return pltpu.make_async_copy(
            bufs[b], out_hbm.at[pl.ds(row, R), :], souts[b])

    for b in range(NBUF):
        in_copy(b, b).start()

    def group(g, _):
        for b in range(NBUF):
            t = g * NBUF + b
            in_wait(b)
            zero_fix(t, b)
            out_copy(t, b).start()
            nxt = t + NBUF

            @pl.when(nxt < NCHUNK)
            def _():
                out_copy(t, b).wait()
                in_copy(nxt, b).start()
        return 0

    lax.fori_loop(0, NCHUNK // NBUF, group, 0)

    for b in range(NBUF):
        out_copy(NCHUNK - NBUF + b, b).wait()


_pos_embed = functools.partial(
    pl.kernel,
    out_type=jax.ShapeDtypeStruct((B * S, D), jnp.float32),
    mesh=plsc.VectorSubcoreMesh(core_axis_name="c", subcore_axis_name="s"),
    scratch_types=(
        [pltpu.VMEM((B, S), jnp.int32)]
        + [pltpu.VMEM((16,), jnp.int32) for _ in range(NBUF)]
        + [pltpu.VMEM((R, D), jnp.float32) for _ in range(NBUF)]
        + [pltpu.SemaphoreType.DMA for _ in range(2 * NBUF)]
    ),
)(_pos_embed_body)


@jax.jit
def kernel(x, pos):
    out = _pos_embed(x.astype(jnp.int32), pos)
    return out.reshape(B, S, D)
